# baseline (speedup 1.0000x reference)
import jax
import jax.numpy as jnp
from jax import lax
from jax.experimental import pallas as pl
from jax.experimental.pallas import tpu as pltpu

_N_BINS = 10


def _calib_body(logits_ref, labels_ref, out_ref, acc_ref):
    i = pl.program_id(0)
    nsteps = pl.num_programs(0)

    @pl.when(i == 0)
    def _init():
        acc_ref[...] = jnp.zeros_like(acc_ref)

    x = logits_ref[...]
    R, C = x.shape
    shifted = x - jnp.float32(1e-07)
    m = jnp.max(shifted, axis=1, keepdims=True)
    col = lax.broadcasted_iota(jnp.int32, (R, C), 1)
    pred = jnp.min(jnp.where(shifted == m, col, C), axis=1)
    labels = labels_ref[0, 0, :]
    correct = (pred == labels).reshape(R, 1)

    bidx = lax.broadcasted_iota(jnp.int32, (1, _N_BINS), 1).astype(jnp.float32)
    lowers = bidx / jnp.float32(10.0)
    uppers = (bidx + 1.0) / jnp.float32(10.0)
    in_bin = (m > lowers) & (m <= uppers)
    cnt = jnp.sum(in_bin.astype(jnp.float32), axis=0)
    asum = jnp.sum((in_bin & correct).astype(jnp.float32), axis=0)
    acc_ref[0, :] += cnt
    acc_ref[1, :] += asum

    @pl.when(i == nsteps - 1)
    def _fin():
        c = acc_ref[0, :]
        s = acc_ref[1, :]
        out_ref[0, :] = jnp.where(c > 0, s / jnp.maximum(c, 1.0), 0.0)


def kernel(logits, labels):
    N, C = logits.shape
    R = 16000
    grid = N // R
    labels3d = labels.reshape(grid, 1, R)
    out = pl.pallas_call(
        _calib_body,
        grid=(grid,),
        in_specs=[
            pl.BlockSpec((R, C), lambda i: (i, 0)),
            pl.BlockSpec((1, 1, R), lambda i: (i, 0, 0)),
        ],
        out_specs=pl.BlockSpec((1, _N_BINS), lambda i: (0, 0)),
        out_shape=jax.ShapeDtypeStruct((1, _N_BINS), jnp.float32),
        scratch_shapes=[pltpu.VMEM((2, _N_BINS), jnp.float32)],
        compiler_params=pltpu.CompilerParams(
            dimension_semantics=("arbitrary",),
        ),
    )(logits, labels3d)
    return out[0]


# speedup vs baseline: 1.6058x; 1.6058x over previous
import jax
import jax.numpy as jnp
from jax import lax
from jax.experimental import pallas as pl
from jax.experimental.pallas import tpu as pltpu

_N_BINS = 10
_C = 32
_G = 8

_BOUNDS = (0.0, 0.1, 0.2, 0.3, 0.4, 0.5, 0.6, 0.7, 0.8, 0.9, 1.0)


def _calib_body(logits_ref, labels_ref, out_ref, acc_ref):
    i = pl.program_id(0)
    nsteps = pl.num_programs(0)

    @pl.when(i == 0)
    def _init():
        acc_ref[...] = jnp.zeros_like(acc_ref)

    x = logits_ref[...]
    R8 = x.shape[0]
    xt = x.T
    s = xt - jnp.float32(1e-07)
    g = s.reshape(_G, _C, R8)

    a = jnp.maximum(
        jnp.maximum(g[:, 0:8], g[:, 8:16]),
        jnp.maximum(g[:, 16:24], g[:, 24:32]),
    )
    m = jnp.concatenate(
        [jnp.max(a[k], axis=0, keepdims=True) for k in range(_G)], axis=0
    )

    sr2 = lax.broadcasted_iota(jnp.int32, (_G * _C, _G), 0)
    gr2 = lax.broadcasted_iota(jnp.int32, (_G * _C, _G), 1)
    bc = ((sr2 // _C) == gr2).astype(jnp.float32)
    m_full = jax.lax.dot_general(
        bc, m, (((1,), (0,)), ((), ())),
        precision=lax.Precision.HIGHEST,
        preferred_element_type=jnp.float32,
    )
    eqf = (s == m_full).astype(jnp.bfloat16)
    sr = lax.broadcasted_iota(jnp.int32, (_G, _G * _C), 1)
    gr = lax.broadcasted_iota(jnp.int32, (_G, _G * _C), 0)
    wbits = jnp.where((sr // _C) == gr, (158 - (sr % _C)) << 23, 0)
    w = lax.bitcast_convert_type(wbits, jnp.float32).astype(jnp.bfloat16)
    sig = jax.lax.dot_general(
        w, eqf, (((1,), (0,)), ((), ())),
        preferred_element_type=jnp.float32,
    )

    labels = labels_ref[0]
    sbits = lax.bitcast_convert_type(sig, jnp.int32)
    cf = ((sbits >> 23) == (158 - labels)).astype(jnp.bfloat16)

    gts = [(m > b).astype(jnp.bfloat16) for b in _BOUNDS]
    gstack = jnp.stack(gts, axis=0)
    astack = gstack * cf[None]
    both = jnp.concatenate([gstack, astack], axis=0).reshape(22 * _G, R8)
    ones = jnp.ones((R8, 1), jnp.bfloat16)
    sums = jax.lax.dot_general(
        both, ones, (((1,), (0,)), ((), ())),
        preferred_element_type=jnp.float32,
    )
    acc_ref[...] += sums

    @pl.when(i == nsteps - 1)
    def _fin():
        tot = jnp.sum(acc_ref[...].reshape(22, _G, 1), axis=1)
        tot = tot.reshape(1, 22)
        c = tot[0:1, 0:10] - tot[0:1, 1:11]
        a = tot[0:1, 11:21] - tot[0:1, 12:22]
        out_ref[...] = jnp.where(c > 0, a / jnp.maximum(c, 1.0), 0.0)


def kernel(logits, labels):
    N, C = logits.shape
    R8 = 5000
    N8 = N // 8
    grid = N8 // R8
    logits8 = logits.reshape(N8, 256)
    labels_t = labels.reshape(grid, R8, 8).transpose(0, 2, 1)
    out = pl.pallas_call(
        _calib_body,
        grid=(grid,),
        in_specs=[
            pl.BlockSpec((R8, 256), lambda i: (i, 0)),
            pl.BlockSpec((1, 8, R8), lambda i: (i, 0, 0)),
        ],
        out_specs=pl.BlockSpec((1, _N_BINS), lambda i: (0, 0)),
        out_shape=jax.ShapeDtypeStruct((1, _N_BINS), jnp.float32),
        scratch_shapes=[pltpu.VMEM((176, 1), jnp.float32)],
        compiler_params=pltpu.CompilerParams(
            dimension_semantics=("arbitrary",),
        ),
    )(logits8, labels_t)
    return out[0]


# speedup vs baseline: 2.0809x; 1.2959x over previous
import jax
import jax.numpy as jnp
from jax import lax
from jax.experimental import pallas as pl
from jax.experimental.pallas import tpu as pltpu

_N_BINS = 10
_C = 32
_G = 8

_BOUNDS = (0.0, 0.1, 0.2, 0.3, 0.4, 0.5, 0.6, 0.7, 0.8, 0.9, 1.0)


def _calib_body(logits_ref, labels_ref, out_ref, acc_ref):
    i = pl.program_id(0)
    nsteps = pl.num_programs(0)

    @pl.when(i == 0)
    def _init():
        acc_ref[...] = jnp.zeros_like(acc_ref)

    xr = logits_ref[...]
    R8 = xr.shape[0] // 8
    xt = xr.T
    s = xt - jnp.float32(1e-07)
    g = jnp.stack([s[:, k * R8:(k + 1) * R8] for k in range(_G)], axis=0)

    a = jnp.maximum(
        jnp.maximum(g[:, 0:8], g[:, 8:16]),
        jnp.maximum(g[:, 16:24], g[:, 24:32]),
    )
    m = jnp.concatenate(
        [jnp.max(a[k], axis=0, keepdims=True) for k in range(_G)], axis=0
    )

    eqf = (g == m[:, None, :]).astype(jnp.bfloat16).reshape(_G * _C, R8)
    sr = lax.broadcasted_iota(jnp.int32, (_G, _G * _C), 1)
    gr = lax.broadcasted_iota(jnp.int32, (_G, _G * _C), 0)
    wbits = jnp.where((sr // _C) == gr, (158 - (sr % _C)) << 23, 0)
    w = lax.bitcast_convert_type(wbits, jnp.float32).astype(jnp.bfloat16)
    sig = jax.lax.dot_general(
        w, eqf, (((1,), (0,)), ((), ())),
        preferred_element_type=jnp.float32,
    )

    labels = labels_ref[0]
    sbits = lax.bitcast_convert_type(sig, jnp.int32)
    cf = ((sbits >> 23) == (158 - labels)).astype(jnp.bfloat16)

    gts = [(m > b).astype(jnp.bfloat16) for b in _BOUNDS]
    gstack = jnp.stack(gts, axis=0)
    astack = gstack * cf[None]
    both = jnp.concatenate([gstack, astack], axis=0).reshape(22 * _G, R8)
    ones = jnp.ones((R8, 1), jnp.bfloat16)
    sums = jax.lax.dot_general(
        both, ones, (((1,), (0,)), ((), ())),
        preferred_element_type=jnp.float32,
    )
    acc_ref[...] += sums

    @pl.when(i == nsteps - 1)
    def _fin():
        tot = jnp.sum(acc_ref[...].reshape(22, _G, 1), axis=1)
        tot = tot.reshape(1, 22)
        c = tot[0:1, 0:10] - tot[0:1, 1:11]
        a = tot[0:1, 11:21] - tot[0:1, 12:22]
        out_ref[...] = jnp.where(c > 0, a / jnp.maximum(c, 1.0), 0.0)


def kernel(logits, labels):
    N, C = logits.shape
    R8 = 2000
    N8 = N // 8
    grid = N8 // R8
    labels_t = labels.reshape(grid, 8, R8)
    out = pl.pallas_call(
        _calib_body,
        grid=(grid,),
        in_specs=[
            pl.BlockSpec((R8 * 8, 32), lambda i: (i, 0)),
            pl.BlockSpec((1, 8, R8), lambda i: (i, 0, 0)),
        ],
        out_specs=pl.BlockSpec((1, _N_BINS), lambda i: (0, 0)),
        out_shape=jax.ShapeDtypeStruct((1, _N_BINS), jnp.float32),
        scratch_shapes=[pltpu.VMEM((176, 1), jnp.float32)],
        compiler_params=pltpu.CompilerParams(
            dimension_semantics=("arbitrary",),
        ),
    )(logits, labels_t)
    return out[0]


# speedup vs baseline: 2.1163x; 1.0170x over previous
import jax
import jax.numpy as jnp
from jax import lax
from jax.experimental import pallas as pl
from jax.experimental.pallas import tpu as pltpu

_N_BINS = 10
_C = 32
_G = 8

_BOUNDS = (0.0, 0.1, 0.2, 0.3, 0.4, 0.5, 0.6, 0.7, 0.8, 0.9, 1.0)


def _calib_body(logits_ref, labels_ref, out_ref, acc_ref):
    i = pl.program_id(0)
    nsteps = pl.num_programs(0)

    @pl.when(i == 0)
    def _init():
        acc_ref[...] = jnp.zeros_like(acc_ref)

    xr = logits_ref[...]
    R8 = xr.shape[0] // 8
    xt = xr.T
    s = xt - jnp.float32(1e-07)
    g = jnp.stack([s[:, k * R8:(k + 1) * R8] for k in range(_G)], axis=0)

    a = jnp.maximum(
        jnp.maximum(g[:, 0:8], g[:, 8:16]),
        jnp.maximum(g[:, 16:24], g[:, 24:32]),
    )
    m = jnp.concatenate(
        [jnp.max(a[k], axis=0, keepdims=True) for k in range(_G)], axis=0
    )

    eqf = (g == m[:, None, :]).astype(jnp.bfloat16).reshape(_G * _C, R8)
    sr = lax.broadcasted_iota(jnp.int32, (_G, _G * _C), 1)
    gr = lax.broadcasted_iota(jnp.int32, (_G, _G * _C), 0)
    wbits = jnp.where((sr // _C) == gr, (158 - (sr % _C)) << 23, 0)
    w = lax.bitcast_convert_type(wbits, jnp.float32).astype(jnp.bfloat16)
    sig = jax.lax.dot_general(
        w, eqf, (((1,), (0,)), ((), ())),
        preferred_element_type=jnp.float32,
    )

    labels = labels_ref[0]
    sbits = lax.bitcast_convert_type(sig, jnp.int32)
    cf = ((sbits >> 23) == (158 - labels)).astype(jnp.bfloat16)

    gts = [(m > b).astype(jnp.bfloat16) for b in _BOUNDS]
    gstack = jnp.stack(gts, axis=0)
    astack = gstack * cf[None]
    both = jnp.concatenate([gstack, astack], axis=0).reshape(22 * _G, R8)
    ones = jnp.ones((R8, 1), jnp.bfloat16)
    sums = jax.lax.dot_general(
        both, ones, (((1,), (0,)), ((), ())),
        preferred_element_type=jnp.float32,
    )
    acc_ref[...] += sums

    @pl.when(i == nsteps - 1)
    def _fin():
        tot = jnp.sum(acc_ref[...].reshape(22, _G, 1), axis=1)
        tot = tot.reshape(1, 22)
        c = tot[0:1, 0:10] - tot[0:1, 1:11]
        a = tot[0:1, 11:21] - tot[0:1, 12:22]
        out_ref[...] = jnp.where(c > 0, a / jnp.maximum(c, 1.0), 0.0)


def kernel(logits, labels):
    N, C = logits.shape
    R8 = 2500
    N8 = N // 8
    grid = N8 // R8
    labels_t = labels.reshape(grid, 8, R8)
    out = pl.pallas_call(
        _calib_body,
        grid=(grid,),
        in_specs=[
            pl.BlockSpec((R8 * 8, 32), lambda i: (i, 0)),
            pl.BlockSpec((1, 8, R8), lambda i: (i, 0, 0)),
        ],
        out_specs=pl.BlockSpec((1, _N_BINS), lambda i: (0, 0)),
        out_shape=jax.ShapeDtypeStruct((1, _N_BINS), jnp.float32),
        scratch_shapes=[pltpu.VMEM((176, 1), jnp.float32)],
        compiler_params=pltpu.CompilerParams(
            dimension_semantics=("arbitrary",),
        ),
    )(logits, labels_t)
    return out[0]


# speedup vs baseline: 2.1303x; 1.0066x over previous
import jax
import jax.numpy as jnp
from jax import lax
from jax.experimental import pallas as pl
from jax.experimental.pallas import tpu as pltpu

_N_BINS = 10
_C = 32
_G = 8

_BOUNDS = (0.0, 0.1, 0.2, 0.3, 0.4, 0.5, 0.6, 0.7, 0.8, 0.9, 1.0)


def _calib_body(logits_ref, labels_ref, out_ref, acc_ref):
    i = pl.program_id(0)
    nsteps = pl.num_programs(0)

    @pl.when(i == 0)
    def _init():
        acc_ref[...] = jnp.zeros_like(acc_ref)

    xr = logits_ref[...]
    R8 = xr.shape[0] // 8
    xt = xr.T
    s = xt - jnp.float32(1e-07)
    g = jnp.stack([s[:, k * R8:(k + 1) * R8] for k in range(_G)], axis=0)

    a = jnp.maximum(
        jnp.maximum(g[:, 0:8], g[:, 8:16]),
        jnp.maximum(g[:, 16:24], g[:, 24:32]),
    )
    m = jnp.concatenate(
        [jnp.max(a[k], axis=0, keepdims=True) for k in range(_G)], axis=0
    )

    eqf = (g == m[:, None, :]).astype(jnp.bfloat16).reshape(_G * _C, R8)
    sr = lax.broadcasted_iota(jnp.int32, (_G, _G * _C), 1)
    gr = lax.broadcasted_iota(jnp.int32, (_G, _G * _C), 0)
    wbits = jnp.where((sr // _C) == gr, (158 - (sr % _C)) << 23, 0)
    w = lax.bitcast_convert_type(wbits, jnp.float32).astype(jnp.bfloat16)
    sig = jax.lax.dot_general(
        w, eqf, (((1,), (0,)), ((), ())),
        preferred_element_type=jnp.float32,
    )

    labels = labels_ref[0]
    sbits = lax.bitcast_convert_type(sig, jnp.int32)
    cf = ((sbits >> 23) == (158 - labels)).astype(jnp.bfloat16)

    gts = [(m > b).astype(jnp.bfloat16) for b in _BOUNDS]
    gstack = jnp.stack(gts, axis=0)
    astack = gstack * cf[None]
    both = jnp.concatenate([gstack, astack], axis=0).reshape(22 * _G, R8)
    ones = jnp.ones((R8, 1), jnp.bfloat16)
    sums = jax.lax.dot_general(
        both, ones, (((1,), (0,)), ((), ())),
        preferred_element_type=jnp.float32,
    )
    acc_ref[...] += sums

    @pl.when(i == nsteps - 1)
    def _fin():
        tot = jnp.sum(acc_ref[...].reshape(22, _G, 1), axis=1)
        tot = tot.reshape(1, 22)
        c = tot[0:1, 0:10] - tot[0:1, 1:11]
        a = tot[0:1, 11:21] - tot[0:1, 12:22]
        out_ref[...] = jnp.where(c > 0, a / jnp.maximum(c, 1.0), 0.0)


def kernel(logits, labels):
    N, C = logits.shape
    R8 = 3125
    N8 = N // 8
    grid = N8 // R8
    labels_t = labels.reshape(grid, 8, R8)
    out = pl.pallas_call(
        _calib_body,
        grid=(grid,),
        in_specs=[
            pl.BlockSpec((R8 * 8, 32), lambda i: (i, 0)),
            pl.BlockSpec((1, 8, R8), lambda i: (i, 0, 0)),
        ],
        out_specs=pl.BlockSpec((1, _N_BINS), lambda i: (0, 0)),
        out_shape=jax.ShapeDtypeStruct((1, _N_BINS), jnp.float32),
        scratch_shapes=[pltpu.VMEM((176, 1), jnp.float32)],
        compiler_params=pltpu.CompilerParams(
            dimension_semantics=("arbitrary",),
        ),
    )(logits, labels_t)
    return out[0]
